# baseline (speedup 1.0000x reference)
import functools

import jax
import jax.numpy as jnp
from jax import lax
from jax.experimental import pallas as pl
from jax.experimental.pallas import tpu as pltpu
from jax.experimental.pallas import tpu_sc as plsc

NINP = 64
NC = 2
NS = 16
NW = NC * NS
G = 128
S = 4
R = G * S


def _gather_body(n_chunks, table_hbm, idx_hbm, out_hbm, idx_v, rows_v, sem):
    cid = lax.axis_index("c")
    sid = lax.axis_index("s")
    wid = sid * NC + cid

    def chunk(g, carry):
        row0 = (wid * n_chunks + g) * R
        crow = (wid * n_chunks + g) * S
        pltpu.sync_copy(idx_hbm.at[pl.ds(crow, S)], idx_v)
        cps = [
            pltpu.async_copy(
                table_hbm.at[idx_v.at[j]], rows_v.at[pl.ds(j * G, G)], sem
            )
            for j in range(S)
        ]
        for cp in cps:
            cp.wait()
        pltpu.sync_copy(rows_v, out_hbm.at[pl.ds(row0, R)])
        return carry

    lax.fori_loop(0, n_chunks, chunk, 0)


def kernel(input, word_embed_weight):
    B = input.shape[0] * input.shape[1]
    assert B % (NW * R) == 0
    n_chunks = B // (NW * R)

    idx2 = input.reshape(B // G, G).astype(jnp.int32)

    mesh = plsc.VectorSubcoreMesh(core_axis_name="c", subcore_axis_name="s")
    k = functools.partial(
        pl.kernel,
        mesh=mesh,
        out_type=jax.ShapeDtypeStruct((B, NINP), jnp.float32),
        scratch_types=[
            pltpu.VMEM((S, G), jnp.int32),
            pltpu.VMEM((R, NINP), jnp.float32),
            pltpu.SemaphoreType.DMA,
        ],
        compiler_params=pltpu.CompilerParams(use_tc_tiling_on_sc=False),
    )(functools.partial(_gather_body, n_chunks))

    out = k(word_embed_weight, idx2)
    return out.reshape(input.shape[0], input.shape[1], NINP)


# speedup vs baseline: 1.0427x; 1.0427x over previous
import functools

import jax
import jax.numpy as jnp
from jax import lax
from jax.experimental import pallas as pl
from jax.experimental.pallas import tpu as pltpu
from jax.experimental.pallas import tpu_sc as plsc

NINP = 64
NC = 2
NS = 16
NW = NC * NS
G = 128
S = 2
R = G * S
NBUF = 4


def _gather_body(n_chunks, table_hbm, idx_hbm, out_hbm, idx_v, rows_v, gsems, osems):
    cid = lax.axis_index("c")
    sid = lax.axis_index("s")
    wid = sid * NC + cid
    crow0 = wid * (n_chunks * S)
    row0 = wid * (n_chunks * R)

    pltpu.sync_copy(idx_hbm.at[pl.ds(crow0, n_chunks * S)], idx_v)

    def fire_gathers(g, b):
        for j in range(S):
            pltpu.async_copy(
                table_hbm.at[idx_v.at[g * S + j]],
                rows_v.at[b].at[pl.ds(j * G, G)],
                gsems[b],
            )

    def drain_gathers(b):
        for j in range(S):
            pltpu.make_async_copy(
                table_hbm.at[pl.ds(0, G)],
                rows_v.at[b].at[pl.ds(j * G, G)],
                gsems[b],
            ).wait()

    def fire_store(g, b):
        pltpu.async_copy(
            rows_v.at[b], out_hbm.at[pl.ds(row0 + g * R, R)], osems[b]
        )

    def drain_store(b):
        pltpu.make_async_copy(
            rows_v.at[b], out_hbm.at[pl.ds(row0, R)], osems[b]
        ).wait()

    n = n_chunks
    fire_gathers(0, 0)
    for b in range(NBUF - 1):
        fire_gathers(b + 1, b + 1)
        drain_gathers(b)
        fire_store(b, b)
    b = NBUF - 1
    drain_store(0)
    fire_gathers(NBUF, 0)
    drain_gathers(b)
    fire_store(b, b)

    def rotation(i, carry):
        for b in range(NBUF):
            g = i * NBUF + b
            b1 = (b + 1) % NBUF
            drain_store(b1)
            fire_gathers(g + 1, b1)
            drain_gathers(b)
            fire_store(g, b)
        return carry

    lax.fori_loop(1, n // NBUF - 1, rotation, 0)

    for b in range(NBUF):
        g = n - NBUF + b
        if b < NBUF - 1:
            b1 = (b + 1) % NBUF
            drain_store(b1)
            fire_gathers(g + 1, b1)
        drain_gathers(b)
        fire_store(g, b)
    for b in range(NBUF):
        drain_store(b)


def kernel(input, word_embed_weight):
    B = input.shape[0] * input.shape[1]
    assert B % (NW * R) == 0
    n_chunks = B // (NW * R)

    idx2 = input.reshape(B // G, G).astype(jnp.int32)

    mesh = plsc.VectorSubcoreMesh(core_axis_name="c", subcore_axis_name="s")
    k = functools.partial(
        pl.kernel,
        mesh=mesh,
        out_type=jax.ShapeDtypeStruct((B, NINP), jnp.float32),
        scratch_types=[
            pltpu.VMEM((n_chunks * S, G), jnp.int32),
            pltpu.VMEM((NBUF, R, NINP), jnp.float32),
            [pltpu.SemaphoreType.DMA] * NBUF,
            [pltpu.SemaphoreType.DMA] * NBUF,
        ],
        compiler_params=pltpu.CompilerParams(use_tc_tiling_on_sc=False),
    )(functools.partial(_gather_body, n_chunks))

    out = k(word_embed_weight, idx2)
    return out.reshape(input.shape[0], input.shape[1], NINP)
